# Initial kernel scaffold; baseline (speedup 1.0000x reference)
import jax
import jax.numpy as jnp
from jax.experimental import pallas as pl


def kernel(features, coors, coors_inv, scale_coors_inv, W_in, b_in, Wp1, bp1, g1, be1, Wp2, bp2, g2, be2, Wp3, bp3, Wo1, bo1, Wo2, bo2):
    raise NotImplementedError("write your pallas kernel here")


# speedup vs baseline: 1.1076x; 1.1076x over previous
import functools

import jax
import jax.numpy as jnp
from jax import lax
from jax.experimental import pallas as pl
from jax.experimental.pallas import tpu as pltpu

N_VOX = 50000
N_PTS = 100000
N_OUT = 20000
C = 256
KEYSPACE = 100 * 50 * 50 * 50

BLK = 2000


def _leaky(x):
    return jnp.where(x >= 0, x, 0.1 * x)


def _row(b):
    return b.reshape(1, -1)


def _mm2_body(x_ref, w1_ref, b1_ref, w2_ref, o_ref):
    h = jnp.dot(x_ref[...], w1_ref[...], preferred_element_type=jnp.float32)
    h = _leaky(h + b1_ref[...])
    o_ref[...] = jnp.dot(h, w2_ref[...], preferred_element_type=jnp.float32)


def _mm2(x, w1, b1, w2):
    m, k = x.shape
    n1, n2 = w1.shape[1], w2.shape[1]
    return pl.pallas_call(
        _mm2_body,
        grid=(m // BLK,),
        in_specs=[
            pl.BlockSpec((BLK, k), lambda i: (i, 0)),
            pl.BlockSpec((k, n1), lambda i: (0, 0)),
            pl.BlockSpec((1, n1), lambda i: (0, 0)),
            pl.BlockSpec((n1, n2), lambda i: (0, 0)),
        ],
        out_specs=pl.BlockSpec((BLK, n2), lambda i: (i, 0)),
        out_shape=jax.ShapeDtypeStruct((m, n2), jnp.float32),
    )(x, w1, b1, w2)


def _stats_update(st_ref, h, w):
    i = pl.program_id(0)
    n = h.shape[1]
    s1 = jnp.sum(h * w, axis=0, keepdims=True)
    s2 = jnp.sum(h * h * w, axis=0, keepdims=True)
    nv = jnp.full((1, n), jnp.sum(w), jnp.float32)
    contrib = jnp.concatenate([s1, s2, nv, jnp.zeros((5, n), jnp.float32)], axis=0)

    @pl.when(i == 0)
    def _():
        st_ref[...] = contrib

    @pl.when(i > 0)
    def _():
        st_ref[...] = st_ref[...] + contrib


def _h1_body(ds_ref, cnt_ref, w1_ref, b1_ref, h_ref, st_ref):
    cnt = cnt_ref[:, :1]
    w = (cnt > 0).astype(jnp.float32)
    ds = ds_ref[...] * (1.0 / jnp.maximum(cnt, 1.0))
    h = _leaky(jnp.dot(ds, w1_ref[...], preferred_element_type=jnp.float32) + b1_ref[...])
    h_ref[...] = h
    _stats_update(st_ref, h, w)


def _h1(ds_sum, cnt16, wp1, bp1):
    m, k = ds_sum.shape
    n = wp1.shape[1]
    return pl.pallas_call(
        _h1_body,
        grid=(m // BLK,),
        in_specs=[
            pl.BlockSpec((BLK, k), lambda i: (i, 0)),
            pl.BlockSpec((BLK, 16), lambda i: (i, 0)),
            pl.BlockSpec((k, n), lambda i: (0, 0)),
            pl.BlockSpec((1, n), lambda i: (0, 0)),
        ],
        out_specs=[
            pl.BlockSpec((BLK, n), lambda i: (i, 0)),
            pl.BlockSpec((8, n), lambda i: (0, 0)),
        ],
        out_shape=[
            jax.ShapeDtypeStruct((m, n), jnp.float32),
            jax.ShapeDtypeStruct((8, n), jnp.float32),
        ],
    )(ds_sum, cnt16, wp1, bp1)


def _bn_apply(h, st, g, b):
    n = st[2:3, :]
    m = st[0:1, :] / n
    v = st[1:2, :] / n - m * m
    return (h - m) * (1.0 / jnp.sqrt(v + 1e-5)) * g + b


def _bnmm_body(h_ref, st_ref, g_ref, be_ref, cnt_ref, w_ref, b_ref, o_ref, sto_ref,
               *, want_stats):
    hn = _bn_apply(h_ref[...], st_ref[...], g_ref[...], be_ref[...])
    h2 = _leaky(jnp.dot(hn, w_ref[...], preferred_element_type=jnp.float32) + b_ref[...])
    o_ref[...] = h2
    if want_stats:
        w = (cnt_ref[:, :1] > 0).astype(jnp.float32)
        _stats_update(sto_ref, h2, w)


def _bnmm_q_body(h_ref, st_ref, g_ref, be_ref, w_ref, b_ref, wq_ref, o_ref):
    hn = _bn_apply(h_ref[...], st_ref[...], g_ref[...], be_ref[...])
    h3 = _leaky(jnp.dot(hn, w_ref[...], preferred_element_type=jnp.float32) + b_ref[...])
    o_ref[...] = jnp.dot(h3, wq_ref[...], preferred_element_type=jnp.float32)


def _h2(h1, st1, g1, be1, cnt16, wp2, bp2):
    m, k = h1.shape
    n = wp2.shape[1]
    body = functools.partial(_bnmm_body, want_stats=True)
    return pl.pallas_call(
        body,
        grid=(m // BLK,),
        in_specs=[
            pl.BlockSpec((BLK, k), lambda i: (i, 0)),
            pl.BlockSpec((8, k), lambda i: (0, 0)),
            pl.BlockSpec((1, k), lambda i: (0, 0)),
            pl.BlockSpec((1, k), lambda i: (0, 0)),
            pl.BlockSpec((BLK, 16), lambda i: (i, 0)),
            pl.BlockSpec((k, n), lambda i: (0, 0)),
            pl.BlockSpec((1, n), lambda i: (0, 0)),
        ],
        out_specs=[
            pl.BlockSpec((BLK, n), lambda i: (i, 0)),
            pl.BlockSpec((8, n), lambda i: (0, 0)),
        ],
        out_shape=[
            jax.ShapeDtypeStruct((m, n), jnp.float32),
            jax.ShapeDtypeStruct((8, n), jnp.float32),
        ],
    )(h1, st1, g1, be1, cnt16, wp2, bp2)


def _h3q(h2, st2, g2, be2, wp3, bp3, wo1b):
    m, k = h2.shape
    n = wp3.shape[1]
    n2 = wo1b.shape[1]
    return pl.pallas_call(
        _bnmm_q_body,
        grid=(m // BLK,),
        in_specs=[
            pl.BlockSpec((BLK, k), lambda i: (i, 0)),
            pl.BlockSpec((8, k), lambda i: (0, 0)),
            pl.BlockSpec((1, k), lambda i: (0, 0)),
            pl.BlockSpec((1, k), lambda i: (0, 0)),
            pl.BlockSpec((k, n), lambda i: (0, 0)),
            pl.BlockSpec((1, n), lambda i: (0, 0)),
            pl.BlockSpec((n, n2), lambda i: (0, 0)),
        ],
        out_specs=pl.BlockSpec((BLK, n2), lambda i: (i, 0)),
        out_shape=jax.ShapeDtypeStruct((m, n2), jnp.float32),
    )(h2, st2, g2, be2, wp3, bp3, wo1b)


def _z_body(a_ref, qg_ref, b1_ref, w2_ref, b2_ref, o_ref):
    u = _leaky(a_ref[...] + qg_ref[...] + b1_ref[...])
    o_ref[...] = jnp.dot(u, w2_ref[...], preferred_element_type=jnp.float32) + b2_ref[...]


def _zk(a, qg, bo1, wo2, bo2):
    m, n = a.shape
    return pl.pallas_call(
        _z_body,
        grid=(m // BLK,),
        in_specs=[
            pl.BlockSpec((BLK, n), lambda i: (i, 0)),
            pl.BlockSpec((BLK, n), lambda i: (i, 0)),
            pl.BlockSpec((1, n), lambda i: (0, 0)),
            pl.BlockSpec((n, n), lambda i: (0, 0)),
            pl.BlockSpec((1, n), lambda i: (0, 0)),
        ],
        out_specs=pl.BlockSpec((BLK, n), lambda i: (i, 0)),
        out_shape=jax.ShapeDtypeStruct((m, n), jnp.float32),
    )(a, qg, bo1, wo2, bo2)


def _div_body(s_ref, c_ref, o_ref):
    c = jnp.maximum(c_ref[:, :1], 1.0)
    o_ref[...] = s_ref[...] * (1.0 / c)


def _divk(vsum, cnt16):
    m, n = vsum.shape
    return pl.pallas_call(
        _div_body,
        grid=(m // BLK,),
        in_specs=[
            pl.BlockSpec((BLK, n), lambda i: (i, 0)),
            pl.BlockSpec((BLK, 16), lambda i: (i, 0)),
        ],
        out_specs=pl.BlockSpec((BLK, n), lambda i: (i, 0)),
        out_shape=jax.ShapeDtypeStruct((m, n), jnp.float32),
    )(vsum, cnt16)


def _labels(coors):
    batch = coors[:, 0]
    c = coors[:, 1:] // 2
    key = ((batch * 50 + c[:, 0]) * 50 + c[:, 1]) * 50 + c[:, 2]
    t = jnp.zeros((KEYSPACE,), jnp.int32)
    t = t.at[key].set(jnp.arange(N_VOX, dtype=jnp.int32))
    return t[key]


def _segsum16(data, idx, num):
    s = jax.ops.segment_sum(data, idx, num_segments=num)
    c = jax.ops.segment_sum(jnp.ones((data.shape[0],), jnp.float32), idx,
                            num_segments=num)
    return s, jnp.tile(c[:, None], (1, 16))


def kernel(features, coors, coors_inv, scale_coors_inv, W_in, b_in, Wp1, bp1,
           g1, be1, Wp2, bp2, g2, be2, Wp3, bp3, Wo1, bo1, Wo2, bo2):
    lab = _labels(coors)
    ds_sum, cnt16 = _segsum16(features, lab, N_VOX)

    wo1a, wo1b = Wo1[:C], Wo1[C:]
    a = _mm2(features, W_in, _row(b_in), wo1a)
    h1, st1 = _h1(ds_sum, cnt16, Wp1, _row(bp1))
    h2, st2 = _h2(h1, st1, _row(g1), _row(be1), cnt16, Wp2, _row(bp2))
    q = _h3q(h2, st2, _row(g2), _row(be2), Wp3, _row(bp3), wo1b)
    qg = q[lab]
    z = _zk(a, qg, _row(bo1), Wo2, _row(bo2))

    lo = z[coors_inv]
    vsum, cnt2 = _segsum16(lo, scale_coors_inv, N_OUT)
    return _divk(vsum, cnt2)


# speedup vs baseline: 2.2019x; 1.9879x over previous
import functools

import jax
import jax.numpy as jnp
from jax import lax
from jax.experimental import pallas as pl
from jax.experimental.pallas import tpu as pltpu
from jax.experimental.pallas import tpu_sc as plsc

NC = 2
NS = 16
L = 16

N_VOX = 50000
N_PTS = 100000
N_OUT = 20000
C = 256
KEYSPACE = 100 * 50 * 50 * 50

BLK = 2000


def _leaky(x):
    return jnp.where(x >= 0, x, 0.1 * x)


def _row(b):
    return b.reshape(1, -1)


def _mm2_body(x_ref, w1_ref, b1_ref, w2_ref, o_ref):
    h = jnp.dot(x_ref[...], w1_ref[...], preferred_element_type=jnp.float32)
    h = _leaky(h + b1_ref[...])
    o_ref[...] = jnp.dot(h, w2_ref[...], preferred_element_type=jnp.float32)


def _mm2(x, w1, b1, w2):
    m, k = x.shape
    n1, n2 = w1.shape[1], w2.shape[1]
    return pl.pallas_call(
        _mm2_body,
        grid=(m // BLK,),
        in_specs=[
            pl.BlockSpec((BLK, k), lambda i: (i, 0)),
            pl.BlockSpec((k, n1), lambda i: (0, 0)),
            pl.BlockSpec((1, n1), lambda i: (0, 0)),
            pl.BlockSpec((n1, n2), lambda i: (0, 0)),
        ],
        out_specs=pl.BlockSpec((BLK, n2), lambda i: (i, 0)),
        out_shape=jax.ShapeDtypeStruct((m, n2), jnp.float32),
    )(x, w1, b1, w2)


def _stats_update(st_ref, h, w):
    i = pl.program_id(0)
    n = h.shape[1]
    s1 = jnp.sum(h * w, axis=0, keepdims=True)
    s2 = jnp.sum(h * h * w, axis=0, keepdims=True)
    nv = jnp.full((1, n), jnp.sum(w), jnp.float32)
    contrib = jnp.concatenate([s1, s2, nv, jnp.zeros((5, n), jnp.float32)], axis=0)

    @pl.when(i == 0)
    def _():
        st_ref[...] = contrib

    @pl.when(i > 0)
    def _():
        st_ref[...] = st_ref[...] + contrib


def _h1_body(ds_ref, cnt_ref, w1_ref, b1_ref, h_ref, st_ref):
    cnt = cnt_ref[:, :1]
    w = (cnt > 0).astype(jnp.float32)
    ds = ds_ref[...] * (1.0 / jnp.maximum(cnt, 1.0))
    h = _leaky(jnp.dot(ds, w1_ref[...], preferred_element_type=jnp.float32) + b1_ref[...])
    h_ref[...] = h
    _stats_update(st_ref, h, w)


def _h1(ds_sum, cnt16, wp1, bp1):
    m, k = ds_sum.shape
    n = wp1.shape[1]
    return pl.pallas_call(
        _h1_body,
        grid=(m // BLK,),
        in_specs=[
            pl.BlockSpec((BLK, k), lambda i: (i, 0)),
            pl.BlockSpec((BLK, 128), lambda i: (i, 0)),
            pl.BlockSpec((k, n), lambda i: (0, 0)),
            pl.BlockSpec((1, n), lambda i: (0, 0)),
        ],
        out_specs=[
            pl.BlockSpec((BLK, n), lambda i: (i, 0)),
            pl.BlockSpec((8, n), lambda i: (0, 0)),
        ],
        out_shape=[
            jax.ShapeDtypeStruct((m, n), jnp.float32),
            jax.ShapeDtypeStruct((8, n), jnp.float32),
        ],
    )(ds_sum, cnt16, wp1, bp1)


def _bn_apply(h, st, g, b):
    n = st[2:3, :]
    m = st[0:1, :] / n
    v = st[1:2, :] / n - m * m
    return (h - m) * (1.0 / jnp.sqrt(v + 1e-5)) * g + b


def _bnmm_body(h_ref, st_ref, g_ref, be_ref, cnt_ref, w_ref, b_ref, o_ref, sto_ref,
               *, want_stats):
    hn = _bn_apply(h_ref[...], st_ref[...], g_ref[...], be_ref[...])
    h2 = _leaky(jnp.dot(hn, w_ref[...], preferred_element_type=jnp.float32) + b_ref[...])
    o_ref[...] = h2
    if want_stats:
        w = (cnt_ref[:, :1] > 0).astype(jnp.float32)
        _stats_update(sto_ref, h2, w)


def _bnmm_q_body(h_ref, st_ref, g_ref, be_ref, w_ref, b_ref, wq_ref, o_ref):
    hn = _bn_apply(h_ref[...], st_ref[...], g_ref[...], be_ref[...])
    h3 = _leaky(jnp.dot(hn, w_ref[...], preferred_element_type=jnp.float32) + b_ref[...])
    o_ref[...] = jnp.dot(h3, wq_ref[...], preferred_element_type=jnp.float32)


def _h2(h1, st1, g1, be1, cnt16, wp2, bp2):
    m, k = h1.shape
    n = wp2.shape[1]
    body = functools.partial(_bnmm_body, want_stats=True)
    return pl.pallas_call(
        body,
        grid=(m // BLK,),
        in_specs=[
            pl.BlockSpec((BLK, k), lambda i: (i, 0)),
            pl.BlockSpec((8, k), lambda i: (0, 0)),
            pl.BlockSpec((1, k), lambda i: (0, 0)),
            pl.BlockSpec((1, k), lambda i: (0, 0)),
            pl.BlockSpec((BLK, 128), lambda i: (i, 0)),
            pl.BlockSpec((k, n), lambda i: (0, 0)),
            pl.BlockSpec((1, n), lambda i: (0, 0)),
        ],
        out_specs=[
            pl.BlockSpec((BLK, n), lambda i: (i, 0)),
            pl.BlockSpec((8, n), lambda i: (0, 0)),
        ],
        out_shape=[
            jax.ShapeDtypeStruct((m, n), jnp.float32),
            jax.ShapeDtypeStruct((8, n), jnp.float32),
        ],
    )(h1, st1, g1, be1, cnt16, wp2, bp2)


def _h3q(h2, st2, g2, be2, wp3, bp3, wo1b):
    m, k = h2.shape
    n = wp3.shape[1]
    n2 = wo1b.shape[1]
    return pl.pallas_call(
        _bnmm_q_body,
        grid=(m // BLK,),
        in_specs=[
            pl.BlockSpec((BLK, k), lambda i: (i, 0)),
            pl.BlockSpec((8, k), lambda i: (0, 0)),
            pl.BlockSpec((1, k), lambda i: (0, 0)),
            pl.BlockSpec((1, k), lambda i: (0, 0)),
            pl.BlockSpec((k, n), lambda i: (0, 0)),
            pl.BlockSpec((1, n), lambda i: (0, 0)),
            pl.BlockSpec((n, n2), lambda i: (0, 0)),
        ],
        out_specs=pl.BlockSpec((BLK, n2), lambda i: (i, 0)),
        out_shape=jax.ShapeDtypeStruct((m, n2), jnp.float32),
    )(h2, st2, g2, be2, wp3, bp3, wo1b)


def _z_body(a_ref, qg_ref, b1_ref, w2_ref, b2_ref, o1_ref, o2_ref):
    u = _leaky(a_ref[...] + qg_ref[...] + b1_ref[...])
    z = jnp.dot(u, w2_ref[...], preferred_element_type=jnp.float32) + b2_ref[...]
    o1_ref[...] = z[:, :128]
    o2_ref[...] = z[:, 128:]


def _zk(a, qg, bo1, wo2, bo2):
    m, n = a.shape
    return pl.pallas_call(
        _z_body,
        grid=(m // BLK,),
        in_specs=[
            pl.BlockSpec((BLK, n), lambda i: (i, 0)),
            pl.BlockSpec((BLK, n), lambda i: (i, 0)),
            pl.BlockSpec((1, n), lambda i: (0, 0)),
            pl.BlockSpec((n, n), lambda i: (0, 0)),
            pl.BlockSpec((1, n), lambda i: (0, 0)),
        ],
        out_specs=[pl.BlockSpec((BLK, 128), lambda i: (i, 0)),
                   pl.BlockSpec((BLK, 128), lambda i: (i, 0))],
        out_shape=[jax.ShapeDtypeStruct((m, 128), jnp.float32),
                   jax.ShapeDtypeStruct((m, 128), jnp.float32)],
    )(a, qg, bo1, wo2, bo2)


def _div_body(s1_ref, s2_ref, c_ref, o_ref):
    r = 1.0 / jnp.maximum(c_ref[:, :1], 1.0)
    o_ref[:, :128] = s1_ref[...] * r
    o_ref[:, 128:] = s2_ref[...] * r


def _divk(vs1, vs2, cnt):
    m = vs1.shape[0]
    return pl.pallas_call(
        _div_body,
        grid=(m // BLK,),
        in_specs=[
            pl.BlockSpec((BLK, 128), lambda i: (i, 0)),
            pl.BlockSpec((BLK, 128), lambda i: (i, 0)),
            pl.BlockSpec((BLK, 128), lambda i: (i, 0)),
        ],
        out_specs=pl.BlockSpec((BLK, 256), lambda i: (i, 0)),
        out_shape=jax.ShapeDtypeStruct((m, 256), jnp.float32),
    )(vs1, vs2, cnt)


def _gsm(parts, gidx_pad, sidx_pad, *, V, S, seg_chunk, acc_rows, passes):
    NP = len(parts)
    P_pad = gidx_pad.shape[0]
    PPT = P_pad // NS
    NG = PPT // L
    SEL = PPT + 144
    n_zc = acc_rows // 8
    n_wf = seg_chunk // 128
    w_rem = seg_chunk - n_wf * 128
    mesh = plsc.VectorSubcoreMesh(**_SC_MESH_KW)

    @functools.partial(
        pl.kernel,
        out_type=[jax.ShapeDtypeStruct((S, 128), jnp.float32)
                  for _ in range(NP + 1)],
        mesh=mesh,
        scratch_types=(
            [pltpu.VMEM((PPT,), jnp.int32),
             pltpu.VMEM((PPT,), jnp.int32),
             pltpu.VMEM((SEL,), jnp.int32),
             pltpu.VMEM((SEL,), jnp.int32),
             pltpu.VMEM((128,), jnp.int32),
             pltpu.VMEM((128,), jnp.int32)]
            + [pltpu.VMEM((128, 128), jnp.float32) for _ in range(NP)]
            + [pltpu.VMEM((128, 128), jnp.float32),
               pltpu.VMEM((8, 128), jnp.float32)]
            + [pltpu.VMEM_SHARED((acc_rows, 128), jnp.float32)
               for _ in range(NP + 1)]
            + [pltpu.SemaphoreType.DMA]
        ),
        compiler_params=pltpu.CompilerParams(needs_layout_passes=False),
    )
    def k(*refs):
        data_hs = refs[:NP]
        gidx_h, sidx_h = refs[NP], refs[NP + 1]
        outs = refs[NP + 2 : 2 * NP + 3]
        sv, gv, selg, sels, gchunk, schunk = refs[2 * NP + 3 : 2 * NP + 9]
        rows = refs[2 * NP + 9 : 3 * NP + 9]
        ones, zbuf = refs[3 * NP + 9], refs[3 * NP + 10]
        accs = refs[3 * NP + 11 : 4 * NP + 12]
        sem = refs[4 * NP + 12]
        ci = lax.axis_index("c")
        si = lax.axis_index("s")
        zeros16 = jnp.zeros((L,), jnp.float32)
        ones16 = jnp.ones((L,), jnp.float32)

        pltpu.sync_copy(sidx_h.at[pl.ds(si * PPT, PPT)], sv)
        pltpu.sync_copy(gidx_h.at[pl.ds(si * PPT, PPT)], gv)

        def init_body(r, _):
            for t in range(128 // L):
                ones[r, pl.ds(t * L, L)] = ones16
            return 0
        lax.fori_loop(0, 128, init_body, 0)

        def zinit_body(r, _):
            for t in range(128 // L):
                zbuf[r, pl.ds(t * L, L)] = zeros16
            return 0
        lax.fori_loop(0, 8, zinit_body, 0)

        for p in range(passes):
            seg_base = (ci * passes + p) * seg_chunk

            def zero_body(kk, _):
                c = si + NS * kk
                @pl.when(c < n_zc)
                def _():
                    for a in accs:
                        pltpu.sync_copy(zbuf, a.at[pl.ds(c * 8, 8)])
                return 0
            lax.fori_loop(0, (n_zc + NS - 1) // NS, zero_body, 0)
            plsc.subcore_barrier()

            def filt(g, nsel):
                svv = sv[pl.ds(g * L, L)]
                gvv = gv[pl.ds(g * L, L)]
                loc = svv - seg_base
                m = (loc >= 0) & (loc < seg_chunk)
                cum = plsc.cumsum(m.astype(jnp.int32))
                pos = nsel + cum - 1
                plsc.store_scatter(selg, [pos], gvv, mask=m)
                plsc.store_scatter(sels, [pos], loc, mask=m)
                return nsel + jnp.max(cum)
            nsel = lax.fori_loop(0, NG, filt, jnp.int32(0))

            iota16 = lax.iota(jnp.int32, L)
            for t in range(8):
                tpos = nsel + t * L + iota16
                plsc.store_scatter(selg, [tpos], jnp.full((L,), -1, jnp.int32))
                plsc.store_scatter(sels, [tpos], jnp.full((L,), -1, jnp.int32))

            nch = (nsel + 127) // 128

            def chunk_body(j, _):
                for t in range(8):
                    gchunk[pl.ds(t * L, L)] = selg[pl.ds(j * 128 + t * L, L)]
                    schunk[pl.ds(t * L, L)] = sels[pl.ds(j * 128 + t * L, L)]
                gi = plsc.Indices(gchunk, ignored_value=-1)
                si_ = plsc.Indices(schunk, ignored_value=-1)
                for d, r in zip(data_hs, rows):
                    pltpu.async_copy(d.at[gi], r, sem).wait()
                for r, a in zip(list(rows) + [ones], accs):
                    pltpu.sync_copy(r, a.at[si_], add=True)
                return 0
            lax.fori_loop(0, nch, chunk_body, 0)
            plsc.subcore_barrier()

            for kk in range((n_wf + NS) // NS):
                c = si + NS * kk
                @pl.when(c < n_wf)
                def _():
                    for a, o in zip(accs, outs):
                        pltpu.sync_copy(a.at[pl.ds(c * 128, 128)],
                                        o.at[pl.ds(seg_base + c * 128, 128)])
                if w_rem:
                    @pl.when(c == n_wf)
                    def _():
                        for a, o in zip(accs, outs):
                            pltpu.sync_copy(
                                a.at[pl.ds(n_wf * 128, w_rem)],
                                o.at[pl.ds(seg_base + n_wf * 128, w_rem)])
            plsc.subcore_barrier()

    res = k(*parts, gidx_pad, sidx_pad)
    return res[:NP], res[NP]


_ROWS_A = 1568
_NGA = _ROWS_A // L
_JW = 112

_SC_MESH_KW = dict(core_axis_name="c", subcore_axis_name="s",
                   num_cores=NC, num_subcores=NS)
_SC_PARAMS = None


def _keys_from_coors(cv, g):
    iota = lax.iota(jnp.int32, L)
    rows = g * L + iota
    flat = rows * 4
    b = plsc.load_gather(cv, [flat])
    x = plsc.load_gather(cv, [flat + 1])
    y = plsc.load_gather(cv, [flat + 2])
    z = plsc.load_gather(cv, [flat + 3])
    key = ((b * 50 + x // 2) * 50 + y // 2) * 50 + z // 2
    return key, rows


def _dedup_scatter(coors):
    mesh = plsc.VectorSubcoreMesh(**_SC_MESH_KW)

    @functools.partial(
        pl.kernel,
        out_type=jax.ShapeDtypeStruct((KEYSPACE,), jnp.int32),
        mesh=mesh,
        scratch_types=[
            pltpu.VMEM((_ROWS_A * 4,), jnp.int32),
            pltpu.VMEM((_ROWS_A // _JW, _JW), jnp.int32),
            pltpu.VMEM((_ROWS_A // _JW, _JW), jnp.int32),
        ],
        compiler_params=pltpu.CompilerParams(needs_layout_passes=False),
    )
    def k(coors_h, t_h, cv, keys2, vals2):
        wid = lax.axis_index("s") * NC + lax.axis_index("c")
        base = jnp.minimum(wid * _ROWS_A, N_VOX - _ROWS_A)
        pltpu.sync_copy(coors_h.at[pl.ds(base * 4, _ROWS_A * 4)], cv)

        def g_body(g, _):
            key, rows = _keys_from_coors(cv, g)
            r, c0 = g // (_JW // L), (g % (_JW // L)) * L
            keys2[r, pl.ds(c0, L)] = key
            vals2[r, pl.ds(c0, L)] = base + rows
            return 0
        lax.fori_loop(0, _NGA, g_body, 0)

        def s_body(j, _):
            pltpu.sync_copy(vals2.at[j], t_h.at[keys2.at[j]])
            return 0
        lax.fori_loop(0, _ROWS_A // _JW, s_body, 0)

    return k(coors)


def _dedup_gather(coors, table):
    mesh = plsc.VectorSubcoreMesh(**_SC_MESH_KW)

    @functools.partial(
        pl.kernel,
        out_type=jax.ShapeDtypeStruct((N_VOX,), jnp.int32),
        mesh=mesh,
        scratch_types=[
            pltpu.VMEM((_ROWS_A * 4,), jnp.int32),
            pltpu.VMEM((_ROWS_A,), jnp.int32),
            pltpu.VMEM((_ROWS_A,), jnp.int32),
            pltpu.SemaphoreType.DMA,
        ],
        compiler_params=pltpu.CompilerParams(needs_layout_passes=False),
    )
    def k(coors_h, t_h, lab_h, cv, keysf, labv, sem):
        wid = lax.axis_index("s") * NC + lax.axis_index("c")
        base = jnp.minimum(wid * _ROWS_A, N_VOX - _ROWS_A)
        pltpu.sync_copy(coors_h.at[pl.ds(base * 4, _ROWS_A * 4)], cv)

        def g_body(g, _):
            key, _rows = _keys_from_coors(cv, g)
            keysf[pl.ds(g * L, L)] = key
            return 0
        lax.fori_loop(0, _NGA, g_body, 0)

        def j_body(j, _):
            pltpu.async_copy(t_h.at[keysf.at[pl.ds(j * _JW, _JW)]],
                             labv.at[pl.ds(j * _JW, _JW)], sem).wait()
            return 0
        lax.fori_loop(0, _ROWS_A // _JW, j_body, 0)
        pltpu.sync_copy(labv, lab_h.at[pl.ds(base, _ROWS_A)])

    return k(coors, table)


def _labels(coors):
    cf = coors.reshape(-1)
    return _dedup_gather(cf, _dedup_scatter(cf))


def _row_gather(src, idx):
    mesh = plsc.VectorSubcoreMesh(**_SC_MESH_KW)

    @functools.partial(
        pl.kernel,
        out_type=jax.ShapeDtypeStruct((N_VOX, C), jnp.float32),
        mesh=mesh,
        scratch_types=[
            pltpu.VMEM((_ROWS_A,), jnp.int32),
            pltpu.VMEM((_JW, C), jnp.float32),
            pltpu.SemaphoreType.DMA,
        ],
        compiler_params=pltpu.CompilerParams(needs_layout_passes=False),
    )
    def k(src_h, idx_h, out_h, idxv, rows, sem):
        wid = lax.axis_index("s") * NC + lax.axis_index("c")
        base = jnp.minimum(wid * _ROWS_A, N_VOX - _ROWS_A)
        pltpu.sync_copy(idx_h.at[pl.ds(base, _ROWS_A)], idxv)

        def j_body(j, _):
            pltpu.async_copy(src_h.at[idxv.at[pl.ds(j * _JW, _JW)]],
                             rows, sem).wait()
            pltpu.sync_copy(rows, out_h.at[pl.ds(base + j * _JW, _JW)])
            return 0
        lax.fori_loop(0, _ROWS_A // _JW, j_body, 0)

    return k(src, idx)


def _pad_idx(idx, n_pad):
    return jnp.concatenate(
        [idx, jnp.full((n_pad - idx.shape[0],), -1, jnp.int32)])


def kernel(features, coors, coors_inv, scale_coors_inv, W_in, b_in, Wp1, bp1,
           g1, be1, Wp2, bp2, g2, be2, Wp3, bp3, Wo1, bo1, Wo2, bo2):
    lab = _labels(coors)
    gidx_b = jnp.concatenate([jnp.arange(N_VOX, dtype=jnp.int32),
                              jnp.zeros((50176 - N_VOX,), jnp.int32)])
    (ds_sum,), cnt16 = _gsm([features], gidx_b, _pad_idx(lab, 50176),
                            V=N_VOX, S=50112,
                            seg_chunk=4176, acc_rows=4224, passes=6)
    ds_sum, cnt16 = ds_sum[:N_VOX], cnt16[:N_VOX]

    wo1a, wo1b = Wo1[:C], Wo1[C:]
    a = _mm2(features, W_in, _row(b_in), wo1a)
    h1, st1 = _h1(ds_sum, cnt16, Wp1, _row(bp1))
    h2, st2 = _h2(h1, st1, _row(g1), _row(be1), cnt16, Wp2, _row(bp2))
    q = _h3q(h2, st2, _row(g2), _row(be2), Wp3, _row(bp3), wo1b)
    qg = _row_gather(q, lab)
    z1, z2 = _zk(a, qg, _row(bo1), Wo2, _row(bo2))

    gidx_d = jnp.concatenate([coors_inv, jnp.zeros((100096 - N_PTS,), jnp.int32)])
    (vs1, vs2), cnt2 = _gsm([z1, z2], gidx_d, _pad_idx(scale_coors_inv, 100096),
                            V=N_VOX, S=21280,
                            seg_chunk=2128, acc_rows=2176, passes=5)
    return _divk(vs1[:N_OUT], vs2[:N_OUT], cnt2[:N_OUT])
